# Initial kernel scaffold; baseline (speedup 1.0000x reference)
import jax
import jax.numpy as jnp
from jax.experimental import pallas as pl


def kernel(x, W_proj, b_proj, embed):
    raise NotImplementedError("write your pallas kernel here")


# speedup vs baseline: 1.2834x; 1.2834x over previous
import functools
import math

import jax
import jax.numpy as jnp
from jax import lax
from jax.experimental import pallas as pl
from jax.experimental.pallas import tpu as pltpu
from jax.experimental.pallas import tpu_sc as plsc

NUM_EMBED = 8192
EMBED_FEATS = 32
IN_FEATS = 192
COMMITMENT_COST = 0.25

NUM_TOKENS = 8192
TOK_BLOCK = 1024
N_TOK_BLOCKS = NUM_TOKENS // TOK_BLOCK
CODE_CHUNK = 1024
N_CODE_CHUNKS = NUM_EMBED // CODE_CHUNK

SC_CORES = 2
SC_SUBCORES = 16
SC_WORKERS = SC_CORES * SC_SUBCORES
TOK_PER_WORKER = NUM_TOKENS // SC_WORKERS


def _project_body(xt_ref, wt_ref, b_ref, o_ref):
    o_ref[...] = jnp.dot(xt_ref[...], wt_ref[...],
                         preferred_element_type=jnp.float32) + b_ref[...]


def _project(x_t, wt, b2):
    return pl.pallas_call(
        _project_body,
        grid=(N_TOK_BLOCKS,),
        in_specs=[
            pl.BlockSpec((TOK_BLOCK, IN_FEATS), lambda i: (i, 0)),
            pl.BlockSpec((IN_FEATS, EMBED_FEATS), lambda i: (0, 0)),
            pl.BlockSpec((1, EMBED_FEATS), lambda i: (0, 0)),
        ],
        out_specs=pl.BlockSpec((TOK_BLOCK, EMBED_FEATS), lambda i: (i, 0)),
        out_shape=jax.ShapeDtypeStruct((NUM_TOKENS, EMBED_FEATS), jnp.float32),
    )(x_t, wt, b2)


def _assign_body(f_ref, f2_ref, e2_ref, embt_ref, qidx_ref, dsum_ref):
    i = pl.program_id(0)
    flat = f_ref[...]
    f2 = f2_ref[...]

    def chunk(j, carry):
        bmin, barg = carry
        e = embt_ref[:, pl.ds(j * CODE_CHUNK, CODE_CHUNK)]
        e2 = e2_ref[:, pl.ds(j * CODE_CHUNK, CODE_CHUNK)]
        s = jnp.dot(flat, e, preferred_element_type=jnp.float32)
        d2 = f2 + e2 - 2.0 * s
        lmin = jnp.min(d2, axis=1, keepdims=True)
        iota = lax.broadcasted_iota(jnp.int32, (TOK_BLOCK, CODE_CHUNK), 1)
        larg = jnp.min(jnp.where(d2 <= lmin, iota, NUM_EMBED), axis=1,
                       keepdims=True) + j * CODE_CHUNK
        take = lmin < bmin
        return jnp.where(take, lmin, bmin), jnp.where(take, larg, barg)

    init = (jnp.full((TOK_BLOCK, 1), jnp.inf, jnp.float32),
            jnp.zeros((TOK_BLOCK, 1), jnp.int32))
    bmin, barg = lax.fori_loop(0, N_CODE_CHUNKS, chunk, init)
    qidx_ref[...] = barg

    @pl.when(i == 0)
    def _():
        dsum_ref[...] = jnp.zeros((1, 1), jnp.float32)

    dsum_ref[...] += jnp.sum(bmin, axis=(0, 1), keepdims=True)


def _assign(flat, f2, e2r, embt):
    return pl.pallas_call(
        _assign_body,
        grid=(N_TOK_BLOCKS,),
        in_specs=[
            pl.BlockSpec((TOK_BLOCK, EMBED_FEATS), lambda i: (i, 0)),
            pl.BlockSpec((TOK_BLOCK, 1), lambda i: (i, 0)),
            pl.BlockSpec((1, NUM_EMBED), lambda i: (0, 0)),
            pl.BlockSpec((EMBED_FEATS, NUM_EMBED), lambda i: (0, 0)),
        ],
        out_specs=[
            pl.BlockSpec((TOK_BLOCK, 1), lambda i: (i, 0)),
            pl.BlockSpec((1, 1), lambda i: (0, 0)),
        ],
        out_shape=[
            jax.ShapeDtypeStruct((NUM_TOKENS, 1), jnp.int32),
            jax.ShapeDtypeStruct((1, 1), jnp.float32),
        ],
    )(flat, f2, e2r, embt)


def _gather_hist_body(qidx2d_hbm, emb_hbm, zq_hbm, counts_hbm,
                      idx_a, idx_b, rows_v, hist_v, sem):
    wid = lax.axis_index("s") * SC_CORES + lax.axis_index("c")
    base = wid * TOK_PER_WORKER

    pltpu.sync_copy(qidx2d_hbm.at[2 * wid], idx_a)
    pltpu.sync_copy(qidx2d_hbm.at[2 * wid + 1], idx_b)

    cp0 = pltpu.async_copy(emb_hbm.at[idx_a], rows_v.at[pl.ds(0, 128)], sem)
    cp1 = pltpu.async_copy(emb_hbm.at[idx_b], rows_v.at[pl.ds(128, 128)], sem)
    cp0.wait()
    cp1.wait()
    pltpu.sync_copy(rows_v, zq_hbm.at[pl.ds(base, TOK_PER_WORKER)])

    def zero(k, c):
        hist_v[pl.ds(pl.multiple_of(k * 16, 16), 16)] = jnp.zeros(
            (16,), jnp.float32)
        return c

    lax.fori_loop(0, NUM_EMBED // 16, zero, 0)
    ones = jnp.ones((16,), jnp.float32)
    for half in (idx_a, idx_b):
        for k in range(128 // 16):
            v = half[pl.ds(k * 16, 16)]
            plsc.addupdate_scatter(hist_v, [v], ones)
    pltpu.sync_copy(hist_v, counts_hbm.at[wid])


def _gather_hist(qidx2d, embed):
    mesh = plsc.VectorSubcoreMesh(core_axis_name="c", subcore_axis_name="s")
    run = functools.partial(
        pl.kernel,
        out_type=(
            jax.ShapeDtypeStruct((NUM_TOKENS, EMBED_FEATS), jnp.float32),
            jax.ShapeDtypeStruct((SC_WORKERS, NUM_EMBED), jnp.float32),
        ),
        mesh=mesh,
        scratch_types=[
            pltpu.VMEM((128,), jnp.int32),
            pltpu.VMEM((128,), jnp.int32),
            pltpu.VMEM((TOK_PER_WORKER, EMBED_FEATS), jnp.float32),
            pltpu.VMEM((NUM_EMBED,), jnp.float32),
            pltpu.SemaphoreType.DMA,
        ],
        compiler_params=pltpu.CompilerParams(needs_layout_passes=False,
                                             use_tc_tiling_on_sc=False),
    )(_gather_hist_body)
    return run(qidx2d, embed)


def _finalize_body(counts_ref, dsum_ref, loss_ref, lp_ref):
    counts = jnp.sum(counts_ref[...], axis=0, keepdims=True)
    probs = counts * (1.0 / NUM_TOKENS)
    lp_ref[...] = -jnp.sum(probs * jnp.log(probs + 1e-10), axis=(0, 1),
                           keepdims=True)
    loss_ref[...] = dsum_ref[...] * (
        COMMITMENT_COST / (NUM_TOKENS * EMBED_FEATS))


def _finalize(counts, dsum):
    return pl.pallas_call(
        _finalize_body,
        out_shape=[
            jax.ShapeDtypeStruct((1, 1), jnp.float32),
            jax.ShapeDtypeStruct((1, 1), jnp.float32),
        ],
    )(counts, dsum)


def kernel(x, W_proj, b_proj, embed):
    B = x.shape[0]
    x_t = x.transpose(0, 3, 2, 1).reshape(NUM_TOKENS, IN_FEATS)
    wt = W_proj.T
    b2 = b_proj.reshape(1, EMBED_FEATS)
    flat = _project(x_t, wt, b2)
    f2 = jnp.sum(flat ** 2, axis=1, keepdims=True)
    e2r = jnp.sum(embed ** 2, axis=1).reshape(1, NUM_EMBED)
    qidx, dsum = _assign(flat, f2, e2r, embed.T)
    zq, counts = _gather_hist(qidx.reshape(64, 128), embed)
    loss2, lp2 = _finalize(counts, dsum)
    z_q_out = zq.reshape(B, 32, 32, EMBED_FEATS).transpose(0, 3, 2, 1)
    kldiv_r = jnp.full((B, 1), math.log(NUM_EMBED) * (NUM_TOKENS / B),
                       jnp.float32)
    return z_q_out, loss2[0, 0], kldiv_r, lp2[0, 0]


# speedup vs baseline: 1.3852x; 1.0793x over previous
import functools
import math

import jax
import jax.numpy as jnp
from jax import lax
from jax.experimental import pallas as pl
from jax.experimental.pallas import tpu as pltpu
from jax.experimental.pallas import tpu_sc as plsc

NUM_EMBED = 8192
EMBED_FEATS = 32
IN_FEATS = 192
COMMITMENT_COST = 0.25

NUM_TOKENS = 8192
TOK_BLOCK = 1024
N_TOK_BLOCKS = NUM_TOKENS // TOK_BLOCK
CODE_CHUNK = 1024
N_CODE_CHUNKS = NUM_EMBED // CODE_CHUNK

SC_CORES = 2
SC_SUBCORES = 16
SC_WORKERS = SC_CORES * SC_SUBCORES
TOK_PER_WORKER = NUM_TOKENS // SC_WORKERS


def _project_body(xt_ref, wt_ref, b_ref, o_ref):
    o_ref[...] = jnp.dot(xt_ref[...], wt_ref[...],
                         preferred_element_type=jnp.float32) + b_ref[...]


def _project(x_t, wt, b2):
    return pl.pallas_call(
        _project_body,
        grid=(N_TOK_BLOCKS,),
        in_specs=[
            pl.BlockSpec((TOK_BLOCK, IN_FEATS), lambda i: (i, 0)),
            pl.BlockSpec((IN_FEATS, EMBED_FEATS), lambda i: (0, 0)),
            pl.BlockSpec((1, EMBED_FEATS), lambda i: (0, 0)),
        ],
        out_specs=pl.BlockSpec((TOK_BLOCK, EMBED_FEATS), lambda i: (i, 0)),
        out_shape=jax.ShapeDtypeStruct((NUM_TOKENS, EMBED_FEATS), jnp.float32),
    )(x_t, wt, b2)


def _assign_body(f_ref, f2_ref, e2_ref, embt_ref, qidx_ref, dsum_ref):
    i = pl.program_id(0)
    flatn = f_ref[...] * (-2.0)
    f2 = f2_ref[...]
    iota = lax.broadcasted_iota(
        jnp.int32, (TOK_BLOCK, CODE_CHUNK), 1).astype(jnp.float32)

    def chunk(j, carry):
        bmin, barg = carry
        e = embt_ref[:, pl.ds(j * CODE_CHUNK, CODE_CHUNK)]
        e2 = e2_ref[:, pl.ds(j * CODE_CHUNK, CODE_CHUNK)]
        s2 = jnp.dot(flatn, e, preferred_element_type=jnp.float32)
        d2 = f2 + e2 + s2
        lmin = jnp.min(d2, axis=1, keepdims=True)
        larg = jnp.min(jnp.where(d2 <= lmin, iota, float(NUM_EMBED)), axis=1,
                       keepdims=True) + jnp.float32(j * CODE_CHUNK)
        take = lmin < bmin
        return jnp.where(take, lmin, bmin), jnp.where(take, larg, barg)

    init = (jnp.full((TOK_BLOCK, 1), jnp.inf, jnp.float32),
            jnp.zeros((TOK_BLOCK, 1), jnp.float32))
    bmin, barg = lax.fori_loop(0, N_CODE_CHUNKS, chunk, init)
    qidx_ref[...] = barg.astype(jnp.int32)

    @pl.when(i == 0)
    def _():
        dsum_ref[...] = jnp.zeros((1, 1), jnp.float32)

    dsum_ref[...] += jnp.sum(bmin, axis=(0, 1), keepdims=True)


def _assign(flat, f2, e2r, embt):
    return pl.pallas_call(
        _assign_body,
        grid=(N_TOK_BLOCKS,),
        in_specs=[
            pl.BlockSpec((TOK_BLOCK, EMBED_FEATS), lambda i: (i, 0)),
            pl.BlockSpec((TOK_BLOCK, 1), lambda i: (i, 0)),
            pl.BlockSpec((1, NUM_EMBED), lambda i: (0, 0)),
            pl.BlockSpec((EMBED_FEATS, NUM_EMBED), lambda i: (0, 0)),
        ],
        out_specs=[
            pl.BlockSpec((TOK_BLOCK, 1), lambda i: (i, 0)),
            pl.BlockSpec((1, 1), lambda i: (0, 0)),
        ],
        out_shape=[
            jax.ShapeDtypeStruct((NUM_TOKENS, 1), jnp.int32),
            jax.ShapeDtypeStruct((1, 1), jnp.float32),
        ],
    )(flat, f2, e2r, embt)


def _gather_hist_body(qidx2d_hbm, emb_hbm, zq_hbm, counts_hbm,
                      idx_a, idx_b, rows_v, hist_v, sem):
    wid = lax.axis_index("s") * SC_CORES + lax.axis_index("c")
    base = wid * TOK_PER_WORKER

    pltpu.sync_copy(qidx2d_hbm.at[2 * wid], idx_a)
    pltpu.sync_copy(qidx2d_hbm.at[2 * wid + 1], idx_b)

    cp0 = pltpu.async_copy(emb_hbm.at[idx_a], rows_v.at[pl.ds(0, 128)], sem)
    cp1 = pltpu.async_copy(emb_hbm.at[idx_b], rows_v.at[pl.ds(128, 128)], sem)
    cp0.wait()
    cp1.wait()
    pltpu.sync_copy(rows_v, zq_hbm.at[pl.ds(base, TOK_PER_WORKER)])

    def zero(k, c):
        hist_v[pl.ds(pl.multiple_of(k * 16, 16), 16)] = jnp.zeros(
            (16,), jnp.float32)
        return c

    lax.fori_loop(0, NUM_EMBED // 16, zero, 0)
    ones = jnp.ones((16,), jnp.float32)
    for half in (idx_a, idx_b):
        for k in range(128 // 16):
            v = half[pl.ds(k * 16, 16)]
            plsc.addupdate_scatter(hist_v, [v], ones)
    pltpu.sync_copy(hist_v, counts_hbm.at[wid])


def _gather_hist(qidx2d, embed):
    mesh = plsc.VectorSubcoreMesh(core_axis_name="c", subcore_axis_name="s")
    run = functools.partial(
        pl.kernel,
        out_type=(
            jax.ShapeDtypeStruct((NUM_TOKENS, EMBED_FEATS), jnp.float32),
            jax.ShapeDtypeStruct((SC_WORKERS, NUM_EMBED), jnp.float32),
        ),
        mesh=mesh,
        scratch_types=[
            pltpu.VMEM((128,), jnp.int32),
            pltpu.VMEM((128,), jnp.int32),
            pltpu.VMEM((TOK_PER_WORKER, EMBED_FEATS), jnp.float32),
            pltpu.VMEM((NUM_EMBED,), jnp.float32),
            pltpu.SemaphoreType.DMA,
        ],
        compiler_params=pltpu.CompilerParams(needs_layout_passes=False,
                                             use_tc_tiling_on_sc=False),
    )(_gather_hist_body)
    return run(qidx2d, embed)


def _finalize_body(counts_ref, dsum_ref, loss_ref, lp_ref):
    counts = jnp.sum(counts_ref[...], axis=0, keepdims=True)
    probs = counts * (1.0 / NUM_TOKENS)
    lp_ref[...] = -jnp.sum(probs * jnp.log(probs + 1e-10), axis=(0, 1),
                           keepdims=True)
    loss_ref[...] = dsum_ref[...] * (
        COMMITMENT_COST / (NUM_TOKENS * EMBED_FEATS))


def _finalize(counts, dsum):
    return pl.pallas_call(
        _finalize_body,
        out_shape=[
            jax.ShapeDtypeStruct((1, 1), jnp.float32),
            jax.ShapeDtypeStruct((1, 1), jnp.float32),
        ],
    )(counts, dsum)


def kernel(x, W_proj, b_proj, embed):
    B = x.shape[0]
    x_t = x.transpose(0, 3, 2, 1).reshape(NUM_TOKENS, IN_FEATS)
    wt = W_proj.T
    b2 = b_proj.reshape(1, EMBED_FEATS)
    flat = _project(x_t, wt, b2)
    f2 = jnp.sum(flat ** 2, axis=1, keepdims=True)
    e2r = jnp.sum(embed ** 2, axis=1).reshape(1, NUM_EMBED)
    qidx, dsum = _assign(flat, f2, e2r, embed.T)
    zq, counts = _gather_hist(qidx.reshape(64, 128), embed)
    loss2, lp2 = _finalize(counts, dsum)
    z_q_out = zq.reshape(B, 32, 32, EMBED_FEATS).transpose(0, 3, 2, 1)
    kldiv_r = jnp.full((B, 1), math.log(NUM_EMBED) * (NUM_TOKENS / B),
                       jnp.float32)
    return z_q_out, loss2[0, 0], kldiv_r, lp2[0, 0]
